# baseline (speedup 1.0000x reference)
import numpy as np
import jax
import jax.numpy as jnp
from jax.experimental import pallas as pl
from jax.experimental.pallas import tpu as pltpu
from jax.experimental.pallas import tpu_sc as plsc

F = 26
V = 100000
D = 16
B = 16384
NIDX = B * F
W = 128
GRID = NIDX // W
OFF_PERIOD = 13

_OFFS_NP = ((np.arange(OFF_PERIOD * W) % F) * V).astype(np.int32).reshape(OFF_PERIOD, W)

def _sc_gather(cat_flat, offs, emb_flat, fo16):
    _vector_mesh = plsc.VectorSubcoreMesh(core_axis_name="core",
                                          subcore_axis_name="subcore")

    @pl.kernel(
        out_type=[
            jax.ShapeDtypeStruct((NIDX, D), jnp.float32),
            jax.ShapeDtypeStruct((GRID, W), jnp.float32),
        ],
        mesh=_vector_mesh,
        scratch_types=[
            pltpu.VMEM((W,), jnp.int32),
            pltpu.VMEM((W,), jnp.int32),
            pltpu.VMEM((W, 16), jnp.float32),
        ],
        compiler_params=pltpu.CompilerParams(use_tc_tiling_on_sc=False,
                                             needs_layout_passes=False),
    )
    def k(cat_hbm, offs_hbm, emb_hbm, fo_hbm, embout_hbm, foout_hbm,
          gbuf, hbuf, fobuf):
        def body(idx_vmem, offs_vmem, embw_vmem, fow_vmem):
            for t in range(W // 16):
                sl = pl.ds(t * 16, 16)
                g = idx_vmem[0, sl] + offs_vmem[0, sl]
                gbuf[sl] = g
                hbuf[sl] = g >> 4
            pltpu.sync_copy(emb_hbm.at[gbuf], embw_vmem)
            pltpu.sync_copy(fo_hbm.at[hbuf], fobuf)
            iota16 = jax.lax.iota(jnp.int32, 16)
            for t in range(W // 16):
                sl = pl.ds(t * 16, 16)
                rows = iota16 + (t * 16)
                cols = gbuf[sl] & 15
                fow_vmem[0, sl] = plsc.load_gather(fobuf, [rows, cols])

        pltpu.emit_pipeline(
            body,
            grid=(GRID,),
            in_specs=[
                pl.BlockSpec((1, W), index_map=lambda i: (0, i)),
                pl.BlockSpec((1, W), index_map=lambda i: (i % OFF_PERIOD, 0)),
            ],
            out_specs=[
                pl.BlockSpec((W, D), index_map=lambda i: (i, 0)),
                pl.BlockSpec((1, W), index_map=lambda i: (i, 0)),
            ],
            core_axis_name=("core", "subcore"),
            dimension_semantics=(pltpu.PARALLEL,),
        )(cat_hbm, offs_hbm, embout_hbm, foout_hbm)

    return k(cat_flat, offs, emb_flat, fo16)


def _dense_body(num_ref, cat_ref, fo_ref, w1, b1r, wn, bnr, wa, wb, bd1r,
                w2, bd2r, w3, bd3r, wo, bor, s_ref, out_ref):
    f32 = jnp.float32
    num = num_ref[...]
    cat = cat_ref[...]
    fo = fo_ref[...]
    first = (jnp.dot(num, w1[...], preferred_element_type=f32) + b1r[...]
             + jnp.sum(fo, axis=1, keepdims=True))
    nemb = jnp.dot(num, wn[...], preferred_element_type=f32) + bnr[...]
    s_mat = s_ref[...]
    sum_e = jnp.dot(cat, s_mat, preferred_element_type=f32) + nemb
    ssq_e = jnp.dot(cat * cat, s_mat, preferred_element_type=f32) + nemb * nemb
    second = 0.5 * jnp.sum(sum_e * sum_e - ssq_e, axis=1, keepdims=True)
    bf = jnp.bfloat16
    h = (jnp.dot(num.astype(bf), wa[...], preferred_element_type=f32)
         + jnp.dot(cat.astype(bf), wb[...], preferred_element_type=f32)
         + bd1r[...])
    h = jnp.maximum(h, 0.0).astype(bf)
    h = jnp.maximum(jnp.dot(h, w2[...], preferred_element_type=f32) + bd2r[...], 0.0).astype(bf)
    h = jnp.maximum(jnp.dot(h, w3[...], preferred_element_type=f32) + bd3r[...], 0.0).astype(bf)
    dnn = jnp.dot(h, wo[...], preferred_element_type=f32) + bor[...]
    out_ref[...] = first + second + dnn


def _tc_dense(num, cat2d, fo2d, w1, b1r, wn, bnr, wa, wb, bd1r, w2, bd2r, w3, bd3r,
              wo, bor, s_mat, bs=2048):
    nblk = B // bs
    full = lambda a: pl.BlockSpec(a.shape, lambda i: (0,) * a.ndim)
    return pl.pallas_call(
        _dense_body,
        grid=(nblk,),
        in_specs=[
            pl.BlockSpec((bs, num.shape[1]), lambda i: (i, 0)),
            pl.BlockSpec((bs, F * D), lambda i: (i, 0)),
            pl.BlockSpec((bs, F), lambda i: (i, 0)),
            full(w1), full(b1r), full(wn), full(bnr), full(wa), full(wb),
            full(bd1r), full(w2), full(bd2r), full(w3), full(bd3r),
            full(wo), full(bor), full(s_mat),
        ],
        out_specs=pl.BlockSpec((bs, 1), lambda i: (i, 0)),
        out_shape=jax.ShapeDtypeStruct((B, 1), jnp.float32),
    )(num, cat2d, fo2d, w1, b1r, wn, bnr, wa, wb, bd1r, w2, bd2r, w3, bd3r,
      wo, bor, s_mat)


def kernel(numerical_features, categorical_features, W1, b1, fo_tables, emb_tables,
           Wn, bn, Wd1, bd1, Wd2, bd2, Wd3, bd3, Wo, bo):
    cat_flat = categorical_features.astype(jnp.int32).reshape(1, NIDX)
    emb_flat = emb_tables.reshape(F * V, D)
    fo16 = fo_tables.reshape(F * V // 16, 16)
    offs = jnp.asarray(_OFFS_NP)
    emb_rows, fo_rows = _sc_gather(cat_flat, offs, emb_flat, fo16)

    s_mat = jnp.asarray(np.tile(np.eye(D, dtype=np.float32), (F, 1)))
    bf = jnp.bfloat16
    out = _tc_dense(
        numerical_features, emb_rows.reshape(B, F * D), fo_rows.reshape(B, F),
        W1, b1.reshape(1, 1), Wn, bn.reshape(1, D),
        Wd1[:13].astype(bf), Wd1[13:].astype(bf), bd1.reshape(1, 512),
        Wd2.astype(bf), bd2.reshape(1, 256), Wd3.astype(bf), bd3.reshape(1, 128),
        Wo.astype(bf), bo.reshape(1, 1), s_mat,
    )
    return out.reshape(B)


# speedup vs baseline: 1.0852x; 1.0852x over previous
import numpy as np
import jax
import jax.numpy as jnp
from jax import lax
from jax.experimental import pallas as pl
from jax.experimental.pallas import tpu as pltpu
from jax.experimental.pallas import tpu_sc as plsc

F = 26
V = 100000
D = 16
B = 16384
RPW = 32
WIN = B // RPW
IDXW = RPW * F
FOCH = 104
NFOCH = IDXW // FOCH


def _sc_gather(cat, emb2, fo16):
    mesh = plsc.VectorSubcoreMesh(core_axis_name="core", subcore_axis_name="subcore")

    @pl.kernel(
        out_type=jax.ShapeDtypeStruct((4, B, 128), jnp.float32),
        mesh=mesh,
        scratch_types=[
            pltpu.VMEM((4, RPW * 8), jnp.int32),
            pltpu.VMEM((IDXW,), jnp.int32),
            pltpu.VMEM((IDXW,), jnp.int32),
            pltpu.VMEM((IDXW, D), jnp.float32),
            pltpu.VMEM((4 * RPW * 8, D), jnp.float32),
            pltpu.SemaphoreType.DMA,
            pltpu.SemaphoreType.DMA,
        ],
        compiler_params=pltpu.CompilerParams(use_tc_tiling_on_sc=False,
                                             needs_layout_passes=False),
    )
    def k(cat_hbm, emb_hbm, fo_hbm, out_hbm, gbuf, gfo, hfo, fobuf, embsc,
          seme, semf):
        embt = emb_hbm
        fot = fo_hbm
        PN = RPW * 8

        def body(idxb, outb):
            iota = lax.iota(jnp.int32, 16)

            for k_ in range(4):
                @pl.loop(0, PN // 16)
                def _(t, k_=k_):
                    p = t * 16 + iota
                    r = p >> 3
                    s = p & 7
                    f = jnp.minimum(s + (8 * k_), F - 1)
                    g = plsc.load_gather(idxb, [r, f]) + f * V
                    gbuf[k_, pl.ds(t * 16, 16)] = g
                for j in range(PN // 128):
                    pltpu.async_copy(
                        embt.at[gbuf.at[k_, pl.ds(j * 128, 128)]],
                        embsc.at[pl.ds(k_ * PN + j * 128, 128)], seme)

            @pl.loop(0, IDXW // 16)
            def _(t):
                q = t * 16 + iota
                r = q // F
                f = q - r * F
                g = plsc.load_gather(idxb, [r, f]) + f * V
                gfo[pl.ds(t * 16, 16)] = g
                hfo[pl.ds(t * 16, 16)] = g >> 4

            for j in range(NFOCH):
                pltpu.async_copy(fot.at[hfo.at[pl.ds(j * FOCH, FOCH)]],
                                 fobuf.at[pl.ds(j * FOCH, FOCH)], semf)

            for _ in range(4 * (PN // 128)):
                pltpu.make_async_copy(embt.at[gbuf.at[0, pl.ds(0, 128)]],
                                      embsc.at[pl.ds(0, 128)], seme).wait()
            for _ in range(NFOCH):
                pltpu.make_async_copy(fot.at[hfo.at[pl.ds(0, FOCH)]],
                                      fobuf.at[pl.ds(0, FOCH)], semf).wait()

            @pl.loop(0, RPW)
            def _(r):
                for k_ in range(4):
                    for c in range(8):
                        outb[k_, r, pl.ds(c * 16, 16)] = embsc[k_ * PN + r * 8 + c, :]
                q1 = r * F + iota
                g1 = plsc.load_gather(gfo, [q1])
                v1 = plsc.load_gather(fobuf, [q1, g1 & (D - 1)])
                q2 = r * F + 16 + jnp.minimum(iota, F - 16 - 1)
                g2 = plsc.load_gather(gfo, [q2])
                v2 = plsc.load_gather(fobuf, [q2, g2 & (D - 1)])
                outb[3, r, pl.ds(32, 16)] = v1
                outb[3, r, pl.ds(48, 16)] = v2

        pltpu.emit_pipeline(
            body,
            grid=(WIN,),
            in_specs=[pl.BlockSpec((RPW, F), index_map=lambda i: (i, 0))],
            out_specs=[pl.BlockSpec((4, RPW, 128), index_map=lambda i: (0, i, 0))],
            core_axis_name=("core", "subcore"),
            dimension_semantics=(pltpu.PARALLEL,),
        )(cat_hbm, out_hbm)

    return k(cat, emb2, fo16)


def _dense_body(num_ref, x_ref, w1, b1r, wn, bnr, wa, wb4, bd1r,
                w2, bd2r, w3, bd3r, wo, bor, s4_ref, sfo_ref, out_ref):
    f32 = jnp.float32
    bf = jnp.bfloat16
    num = num_ref[...]
    xs = x_ref[...]
    s4 = s4_ref[...]
    nemb = jnp.dot(num, wn[...], preferred_element_type=f32) + bnr[...]
    sum_e = nemb
    ssq_e = nemb * nemb
    for k_ in range(4):
        xk = xs[k_]
        sum_e = sum_e + jnp.dot(xk, s4[k_], preferred_element_type=f32)
        ssq_e = ssq_e + jnp.dot(xk * xk, s4[k_], preferred_element_type=f32)
    second = 0.5 * jnp.sum(sum_e * sum_e - ssq_e, axis=1, keepdims=True)
    fo_sum = jnp.dot(xs[3], sfo_ref[...], preferred_element_type=f32)
    first = jnp.dot(num, w1[...], preferred_element_type=f32) + b1r[...] + fo_sum
    h = jnp.dot(num.astype(bf), wa[...], preferred_element_type=f32) + bd1r[...]
    for k_ in range(4):
        h = h + jnp.dot(xs[k_].astype(bf), wb4[...][k_], preferred_element_type=f32)
    h = jnp.maximum(h, 0.0).astype(bf)
    h = jnp.maximum(jnp.dot(h, w2[...], preferred_element_type=f32) + bd2r[...], 0.0).astype(bf)
    h = jnp.maximum(jnp.dot(h, w3[...], preferred_element_type=f32) + bd3r[...], 0.0).astype(bf)
    dnn = jnp.dot(h, wo[...], preferred_element_type=f32) + bor[...]
    out_ref[...] = (first + second + dnn)[:, 0]


def _tc_dense(num, x4, w1, b1r, wn, bnr, wa, wb4, bd1r, w2, bd2r, w3, bd3r,
              wo, bor, s4, sfo, bs=2048):
    nblk = B // bs
    full = lambda a: pl.BlockSpec(a.shape, lambda i: (0,) * a.ndim)
    return pl.pallas_call(
        _dense_body,
        grid=(nblk,),
        in_specs=[
            pl.BlockSpec((bs, 13), lambda i: (i, 0)),
            pl.BlockSpec((4, bs, 128), lambda i: (0, i, 0)),
            full(w1), full(b1r), full(wn), full(bnr), full(wa), full(wb4),
            full(bd1r), full(w2), full(bd2r), full(w3), full(bd3r),
            full(wo), full(bor), full(s4), full(sfo),
        ],
        out_specs=pl.BlockSpec((bs,), lambda i: (i,)),
        out_shape=jax.ShapeDtypeStruct((B,), jnp.float32),
    )(num, x4, w1, b1r, wn, bnr, wa, wb4, bd1r, w2, bd2r, w3, bd3r, wo, bor,
      s4, sfo)


def _selectors():
    s4 = np.zeros((4, 128, D), np.float32)
    eye = np.eye(D, dtype=np.float32)
    for k_ in range(3):
        s4[k_] = np.tile(eye, (8, 1))
    s4[3, :32] = np.tile(eye, (2, 1))
    sfo = np.zeros((128, 1), np.float32)
    sfo[32:32 + F] = 1.0
    return s4, sfo


_S4_NP, _SFO_NP = _selectors()


def kernel(numerical_features, categorical_features, W1, b1, fo_tables, emb_tables,
           Wn, bn, Wd1, bd1, Wd2, bd2, Wd3, bd3, Wo, bo):
    bf = jnp.bfloat16
    cat = categorical_features.astype(jnp.int32)
    emb2 = emb_tables.reshape(F * V, D)
    fo16 = jnp.concatenate(
        [fo_tables.reshape(-1), jnp.zeros(960, jnp.float32)]).reshape(-1, D)
    x4 = _sc_gather(cat, emb2, fo16)

    wb_pad = jnp.concatenate(
        [Wd1[13:], jnp.zeros((512 - (429 - 13), 512), jnp.float32)], axis=0)
    wb4 = wb_pad.astype(bf).reshape(4, 128, 512)
    out = _tc_dense(
        numerical_features, x4,
        W1, b1.reshape(1, 1), Wn, bn.reshape(1, D),
        Wd1[:13].astype(bf), wb4, bd1.reshape(1, 512),
        Wd2.astype(bf), bd2.reshape(1, 256), Wd3.astype(bf), bd3.reshape(1, 128),
        Wo.astype(bf), bo.reshape(1, 1), jnp.asarray(_S4_NP), jnp.asarray(_SFO_NP),
    )
    return out


# speedup vs baseline: 1.0934x; 1.0076x over previous
import numpy as np
import jax
import jax.numpy as jnp
from jax import lax
from jax.experimental import pallas as pl
from jax.experimental.pallas import tpu as pltpu
from jax.experimental.pallas import tpu_sc as plsc

F = 26
V = 100000
D = 16
B = 16384
RPW = 32
WIN = B // RPW
IDXW = RPW * F
FOCH = 104
NFOCH = IDXW // FOCH


def _sc_gather(cat, emb3, fo16):
    mesh = plsc.VectorSubcoreMesh(core_axis_name="core", subcore_axis_name="subcore")

    @pl.kernel(
        out_type=jax.ShapeDtypeStruct((4, B, 128), jnp.float32),
        mesh=mesh,
        scratch_types=[
            pltpu.VMEM((F, RPW), jnp.int32),
            pltpu.VMEM((IDXW,), jnp.int32),
            pltpu.VMEM((IDXW,), jnp.int32),
            pltpu.VMEM((IDXW, D), jnp.float32),
            pltpu.VMEM((F, RPW, D), jnp.float32),
            pltpu.SemaphoreType.DMA,
            pltpu.SemaphoreType.DMA,
        ],
        compiler_params=pltpu.CompilerParams(use_tc_tiling_on_sc=False,
                                             needs_layout_passes=False),
    )
    def k(cat_hbm, emb_hbm, fo_hbm, out_hbm, vbuf, gfo, hfo, fobuf, embf,
          seme, semf):
        fot = fo_hbm

        def body(idxb, outb):
            iota = lax.iota(jnp.int32, 16)

            for f in range(F):
                for half in range(RPW // 16):
                    vbuf[f, pl.ds(half * 16, 16)] = plsc.load_gather(
                        idxb, [iota + half * 16, jnp.full((16,), f, jnp.int32)])
                pltpu.async_copy(emb_hbm.at[f].at[vbuf.at[f]], embf.at[f], seme)

            @pl.loop(0, IDXW // 16)
            def _(t):
                q = t * 16 + iota
                r = q // F
                f = q - r * F
                g = plsc.load_gather(idxb, [r, f]) + f * V
                gfo[pl.ds(t * 16, 16)] = g
                hfo[pl.ds(t * 16, 16)] = g >> 4

            for j in range(NFOCH):
                pltpu.async_copy(fot.at[hfo.at[pl.ds(j * FOCH, FOCH)]],
                                 fobuf.at[pl.ds(j * FOCH, FOCH)], semf)

            for _ in range(F):
                pltpu.make_async_copy(emb_hbm.at[0].at[vbuf.at[0]],
                                      embf.at[0], seme).wait()
            for _ in range(NFOCH):
                pltpu.make_async_copy(fot.at[hfo.at[pl.ds(0, FOCH)]],
                                      fobuf.at[pl.ds(0, FOCH)], semf).wait()

            zero16 = jnp.zeros((16,), jnp.float32)
            @pl.loop(0, RPW)
            def _(r):
                for k_ in range(3):
                    for c in range(8):
                        outb[k_, r, pl.ds(c * 16, 16)] = embf[8 * k_ + c, r, :]
                outb[3, r, pl.ds(0, 16)] = embf[24, r, :]
                outb[3, r, pl.ds(16, 16)] = embf[25, r, :]
                q1 = r * F + iota
                g1 = plsc.load_gather(gfo, [q1])
                v1 = plsc.load_gather(fobuf, [q1, g1 & (D - 1)])
                q2 = r * F + 16 + jnp.minimum(iota, F - 16 - 1)
                g2 = plsc.load_gather(gfo, [q2])
                v2 = plsc.load_gather(fobuf, [q2, g2 & (D - 1)])
                outb[3, r, pl.ds(32, 16)] = v1
                outb[3, r, pl.ds(48, 16)] = v2
                for c in range(4, 8):
                    outb[3, r, pl.ds(c * 16, 16)] = zero16

        pltpu.emit_pipeline(
            body,
            grid=(WIN,),
            in_specs=[pl.BlockSpec((RPW, F), index_map=lambda i: (i, 0))],
            out_specs=[pl.BlockSpec((4, RPW, 128), index_map=lambda i: (0, i, 0))],
            core_axis_name=("core", "subcore"),
            dimension_semantics=(pltpu.PARALLEL,),
        )(cat_hbm, out_hbm)

    return k(cat, emb3, fo16)


def _dense_body(num_ref, x_ref, w1, b1r, wn, bnr, wa, wb4, bd1r,
                w2, bd2r, w3, bd3r, wo, bor, s4_ref, sfo_ref, out_ref):
    f32 = jnp.float32
    bf = jnp.bfloat16
    num = num_ref[...]
    xs = x_ref[...]
    s4 = s4_ref[...]
    nemb = jnp.dot(num, wn[...], preferred_element_type=f32) + bnr[...]
    sum_e = nemb
    ssq_e = nemb * nemb
    for k_ in range(4):
        xk = xs[k_]
        sum_e = sum_e + jnp.dot(xk, s4[k_], preferred_element_type=f32)
        ssq_e = ssq_e + jnp.dot(xk * xk, s4[k_], preferred_element_type=f32)
    second = 0.5 * jnp.sum(sum_e * sum_e - ssq_e, axis=1, keepdims=True)
    fo_sum = jnp.dot(xs[3], sfo_ref[...], preferred_element_type=f32)
    first = jnp.dot(num, w1[...], preferred_element_type=f32) + b1r[...] + fo_sum
    h = jnp.dot(num.astype(bf), wa[...], preferred_element_type=f32) + bd1r[...]
    for k_ in range(4):
        h = h + jnp.dot(xs[k_].astype(bf), wb4[...][k_], preferred_element_type=f32)
    h = jnp.maximum(h, 0.0).astype(bf)
    h = jnp.maximum(jnp.dot(h, w2[...], preferred_element_type=f32) + bd2r[...], 0.0).astype(bf)
    h = jnp.maximum(jnp.dot(h, w3[...], preferred_element_type=f32) + bd3r[...], 0.0).astype(bf)
    dnn = jnp.dot(h, wo[...], preferred_element_type=f32) + bor[...]
    out_ref[...] = (first + second + dnn)[:, 0]


def _tc_dense(num, x4, w1, b1r, wn, bnr, wa, wb4, bd1r, w2, bd2r, w3, bd3r,
              wo, bor, s4, sfo, bs=2048):
    nblk = B // bs
    full = lambda a: pl.BlockSpec(a.shape, lambda i: (0,) * a.ndim)
    return pl.pallas_call(
        _dense_body,
        grid=(nblk,),
        in_specs=[
            pl.BlockSpec((bs, 13), lambda i: (i, 0)),
            pl.BlockSpec((4, bs, 128), lambda i: (0, i, 0)),
            full(w1), full(b1r), full(wn), full(bnr), full(wa), full(wb4),
            full(bd1r), full(w2), full(bd2r), full(w3), full(bd3r),
            full(wo), full(bor), full(s4), full(sfo),
        ],
        out_specs=pl.BlockSpec((bs,), lambda i: (i,)),
        out_shape=jax.ShapeDtypeStruct((B,), jnp.float32),
    )(num, x4, w1, b1r, wn, bnr, wa, wb4, bd1r, w2, bd2r, w3, bd3r, wo, bor,
      s4, sfo)


def _selectors():
    s4 = np.zeros((4, 128, D), np.float32)
    eye = np.eye(D, dtype=np.float32)
    for k_ in range(3):
        s4[k_] = np.tile(eye, (8, 1))
    s4[3, :32] = np.tile(eye, (2, 1))
    sfo = np.zeros((128, 1), np.float32)
    sfo[32:32 + F] = 1.0
    return s4, sfo


_S4_NP, _SFO_NP = _selectors()


def kernel(numerical_features, categorical_features, W1, b1, fo_tables, emb_tables,
           Wn, bn, Wd1, bd1, Wd2, bd2, Wd3, bd3, Wo, bo):
    bf = jnp.bfloat16
    cat = categorical_features.astype(jnp.int32)
    fo16 = jnp.concatenate(
        [fo_tables.reshape(-1), jnp.zeros(960, jnp.float32)]).reshape(-1, D)
    x4 = _sc_gather(cat, emb_tables, fo16)

    wb_pad = jnp.concatenate(
        [Wd1[13:], jnp.zeros((512 - (429 - 13), 512), jnp.float32)], axis=0)
    wb4 = wb_pad.astype(bf).reshape(4, 128, 512)
    out = _tc_dense(
        numerical_features, x4,
        W1, b1.reshape(1, 1), Wn, bn.reshape(1, D),
        Wd1[:13].astype(bf), wb4, bd1.reshape(1, 512),
        Wd2.astype(bf), bd2.reshape(1, 256), Wd3.astype(bf), bd3.reshape(1, 128),
        Wo.astype(bf), bo.reshape(1, 1), jnp.asarray(_S4_NP), jnp.asarray(_SFO_NP),
    )
    return out
